# baseline (speedup 1.0000x reference)
import jax
import jax.numpy as jnp
from jax import lax
from jax.experimental import pallas as pl
from jax.experimental.pallas import tpu as pltpu
from jax.experimental.pallas import tpu_sc as plsc

BSZ, N_HEADS, MAX_SEQ_LEN, HEAD_DIM = 8, 16, 4096, 64
SEQLEN = 16
BH = BSZ * N_HEADS
NW = 32
LROWS = BH * MAX_SEQ_LEN * HEAD_DIM // 128
ROWS_W = LROWS // NW
CH = 256
NCH = ROWS_W // CH
NBUF = 3


def _in_copy(i, w, c_ref, buf, insem):
    return pltpu.make_async_copy(
        c_ref.at[pl.ds(w * ROWS_W + i * CH, CH)],
        buf.at[i % NBUF],
        insem.at[i % NBUF],
    )


def _out_copy(i, w, o_ref, buf, outsem):
    return pltpu.make_async_copy(
        buf.at[i % NBUF],
        o_ref.at[pl.ds(w * ROWS_W + i * CH, CH)],
        outsem.at[i % NBUF],
    )


def _sc_body(c_ref, o_ref, buf, insem, outsem):
    w = lax.axis_index("s") * 2 + lax.axis_index("c")
    for i in range(NBUF):
        _in_copy(i, w, c_ref, buf, insem).start()
    for i in range(NCH):
        _in_copy(i, w, c_ref, buf, insem).wait()
        _out_copy(i, w, o_ref, buf, outsem).start()
        j = i - (NBUF - 1)
        if j >= 0 and j + NBUF < NCH:
            _out_copy(j, w, o_ref, buf, outsem).wait()
            _in_copy(j + NBUF, w, c_ref, buf, insem).start()
    for j in range(NCH - NBUF, NCH):
        _out_copy(j, w, o_ref, buf, outsem).wait()


def _sc_copy(c2):
    mesh = plsc.VectorSubcoreMesh(core_axis_name="c", subcore_axis_name="s")
    return pl.kernel(
        _sc_body,
        mesh=mesh,
        out_type=jax.ShapeDtypeStruct((LROWS, 128), jnp.float32),
        scratch_types=[
            pltpu.VMEM((NBUF, CH, 128), jnp.float32),
            pltpu.SemaphoreType.DMA((NBUF,)),
            pltpu.SemaphoreType.DMA((NBUF,)),
        ],
    )(c2)


def _upd_body(pos_ref, prev_ref, u_ref, o_ref, sem):
    del prev_ref
    p = pos_ref[0]
    cp = pltpu.make_async_copy(u_ref, o_ref.at[:, pl.ds(p, SEQLEN), :], sem)
    cp.start()
    cp.wait()


def _scatter_update(copied, u3, pos):
    return pl.pallas_call(
        _upd_body,
        grid_spec=pltpu.PrefetchScalarGridSpec(
            num_scalar_prefetch=1,
            grid=(1,),
            in_specs=[
                pl.BlockSpec(memory_space=pl.ANY),
                pl.BlockSpec((BH, SEQLEN, HEAD_DIM), lambda i, p: (0, 0, 0)),
            ],
            out_specs=pl.BlockSpec(memory_space=pl.ANY),
            scratch_shapes=[pltpu.SemaphoreType.DMA],
        ),
        out_shape=jax.ShapeDtypeStruct((BH, MAX_SEQ_LEN, HEAD_DIM), jnp.float32),
        input_output_aliases={1: 0},
    )(pos, copied, u3)


def kernel(cache, update, pos):
    c2 = cache.reshape(LROWS, 128)
    u3 = update.reshape(BH, SEQLEN, HEAD_DIM)
    copied = _sc_copy(c2).reshape(BH, MAX_SEQ_LEN, HEAD_DIM)
    out = _scatter_update(copied, u3, pos)
    return out.reshape(BSZ, N_HEADS, MAX_SEQ_LEN, HEAD_DIM)


# speedup vs baseline: 1.0014x; 1.0014x over previous
import jax
import jax.numpy as jnp
from jax import lax
from jax.experimental import pallas as pl
from jax.experimental.pallas import tpu as pltpu
from jax.experimental.pallas import tpu_sc as plsc

BSZ, N_HEADS, MAX_SEQ_LEN, HEAD_DIM = 8, 16, 4096, 64
SEQLEN = 16
BH = BSZ * N_HEADS
NW = 32
LROWS = BH * MAX_SEQ_LEN * HEAD_DIM // 128
ROWS_W = LROWS // NW
CH = 256
NCH = ROWS_W // CH
NBUF = 3


def _in_copy(i, w, c_ref, buf, insem):
    return pltpu.make_async_copy(
        c_ref.at[pl.ds(w * ROWS_W + i * CH, CH)],
        buf.at[i % NBUF],
        insem.at[i % NBUF],
    )


def _out_copy(i, w, o_ref, buf, outsem):
    return pltpu.make_async_copy(
        buf.at[i % NBUF],
        o_ref.at[pl.ds(w * ROWS_W + i * CH, CH)],
        outsem.at[i % NBUF],
    )


def _sc_body(c_ref, o_ref, buf, insem, outsem):
    w = lax.axis_index("s") * 2 + lax.axis_index("c")
    for i in range(NBUF):
        _in_copy(i, w, c_ref, buf, insem).start()
    for i in range(NCH):
        _in_copy(i, w, c_ref, buf, insem).wait()
        _out_copy(i, w, o_ref, buf, outsem).start()
        j = i - (NBUF - 1)
        if j >= 0 and j + NBUF < NCH:
            _out_copy(j, w, o_ref, buf, outsem).wait()
            _in_copy(j + NBUF, w, c_ref, buf, insem).start()
    for j in range(NCH - NBUF, NCH):
        _out_copy(j, w, o_ref, buf, outsem).wait()


def _sc_copy(c2):
    mesh = plsc.VectorSubcoreMesh(core_axis_name="c", subcore_axis_name="s")
    return pl.kernel(
        _sc_body,
        mesh=mesh,
        out_type=jax.ShapeDtypeStruct((LROWS, 128), jnp.float32),
        scratch_types=[
            pltpu.VMEM((NBUF, CH, 128), jnp.float32),
            pltpu.SemaphoreType.DMA((NBUF,)),
            pltpu.SemaphoreType.DMA((NBUF,)),
        ],
        compiler_params=pltpu.CompilerParams(use_tc_tiling_on_sc=True),
    )(c2)


def _upd_body(pos_ref, prev_ref, u_ref, o_ref, sem):
    del prev_ref
    p = pos_ref[0]
    cp = pltpu.make_async_copy(u_ref, o_ref.at[:, pl.ds(p, SEQLEN), :], sem)
    cp.start()
    cp.wait()


def _scatter_update(copied, u3, pos):
    return pl.pallas_call(
        _upd_body,
        grid_spec=pltpu.PrefetchScalarGridSpec(
            num_scalar_prefetch=1,
            grid=(1,),
            in_specs=[
                pl.BlockSpec(memory_space=pl.ANY),
                pl.BlockSpec((BH, SEQLEN, HEAD_DIM), lambda i, p: (0, 0, 0)),
            ],
            out_specs=pl.BlockSpec(memory_space=pl.ANY),
            scratch_shapes=[pltpu.SemaphoreType.DMA],
        ),
        out_shape=jax.ShapeDtypeStruct((BH, MAX_SEQ_LEN, HEAD_DIM), jnp.float32),
        input_output_aliases={1: 0},
    )(pos, copied, u3)


def kernel(cache, update, pos):
    c2 = cache.reshape(LROWS, 128)
    u3 = update.reshape(BH, SEQLEN, HEAD_DIM)
    copied = _sc_copy(c2).reshape(BH, MAX_SEQ_LEN, HEAD_DIM)
    out = _scatter_update(copied, u3, pos)
    return out.reshape(BSZ, N_HEADS, MAX_SEQ_LEN, HEAD_DIM)


# speedup vs baseline: 3.4066x; 3.4019x over previous
import jax
import jax.numpy as jnp
from jax import lax
from jax.experimental import pallas as pl
from jax.experimental.pallas import tpu as pltpu
from jax.experimental.pallas import tpu_sc as plsc

BSZ, N_HEADS, MAX_SEQ_LEN, HEAD_DIM = 8, 16, 4096, 64
SEQLEN = 16
BH = BSZ * N_HEADS
NW = 32
LROWS = BH * MAX_SEQ_LEN * HEAD_DIM // 128
ROWS_W = LROWS // NW
CH = 256
NCH = ROWS_W // CH
NBUF = 3


CHW = CH * 128


def _in_copy(i, w, c_ref, buf, insem):
    return pltpu.make_async_copy(
        c_ref.at[pl.ds((w * NCH + i) * CHW, CHW)],
        buf.at[i % NBUF],
        insem.at[i % NBUF],
    )


def _out_copy(i, w, o_ref, buf, outsem):
    return pltpu.make_async_copy(
        buf.at[i % NBUF],
        o_ref.at[pl.ds((w * NCH + i) * CHW, CHW)],
        outsem.at[i % NBUF],
    )


def _sc_body(c_ref, o_ref, buf, insem, outsem):
    w = lax.axis_index("s") * 2 + lax.axis_index("c")
    for i in range(NBUF):
        _in_copy(i, w, c_ref, buf, insem).start()
    for i in range(NCH):
        _in_copy(i, w, c_ref, buf, insem).wait()
        _out_copy(i, w, o_ref, buf, outsem).start()
        j = i - (NBUF - 1)
        if j >= 0 and j + NBUF < NCH:
            _out_copy(j, w, o_ref, buf, outsem).wait()
            _in_copy(j + NBUF, w, c_ref, buf, insem).start()
    for j in range(NCH - NBUF, NCH):
        _out_copy(j, w, o_ref, buf, outsem).wait()


def _sc_copy(c2):
    mesh = plsc.VectorSubcoreMesh(core_axis_name="c", subcore_axis_name="s")
    return pl.kernel(
        _sc_body,
        mesh=mesh,
        out_type=jax.ShapeDtypeStruct((LROWS * 128,), jnp.float32),
        scratch_types=[
            pltpu.VMEM((NBUF, CHW), jnp.float32),
            pltpu.SemaphoreType.DMA((NBUF,)),
            pltpu.SemaphoreType.DMA((NBUF,)),
        ],
    )(c2)


def _upd_body(pos_ref, prev_ref, u_ref, o_ref, sem):
    del prev_ref
    p = pos_ref[0]
    cp = pltpu.make_async_copy(u_ref, o_ref.at[:, pl.ds(p, SEQLEN), :], sem)
    cp.start()
    cp.wait()


def _scatter_update(copied, u3, pos):
    return pl.pallas_call(
        _upd_body,
        grid_spec=pltpu.PrefetchScalarGridSpec(
            num_scalar_prefetch=1,
            grid=(1,),
            in_specs=[
                pl.BlockSpec(memory_space=pl.ANY),
                pl.BlockSpec((BH, SEQLEN, HEAD_DIM), lambda i, p: (0, 0, 0)),
            ],
            out_specs=pl.BlockSpec(memory_space=pl.ANY),
            scratch_shapes=[pltpu.SemaphoreType.DMA],
        ),
        out_shape=jax.ShapeDtypeStruct((BH, MAX_SEQ_LEN, HEAD_DIM), jnp.float32),
        input_output_aliases={1: 0},
    )(pos, copied, u3)


def kernel(cache, update, pos):
    c3 = cache.reshape(BH, MAX_SEQ_LEN, HEAD_DIM)
    u3 = update.reshape(BH, SEQLEN, HEAD_DIM)
    out = _scatter_update(c3, u3, pos)
    return out.reshape(BSZ, N_HEADS, MAX_SEQ_LEN, HEAD_DIM)
